# baseline (speedup 1.0000x reference)
import functools

import jax
import jax.numpy as jnp
from jax import lax
from jax.experimental import pallas as pl
from jax.experimental.pallas import tpu as pltpu
from jax.experimental.pallas import tpu_sc as plsc

_NC = 2
_NS = 16
_NW = _NC * _NS
_CH = 128


def _ceil_to(x, m):
    return (x + m - 1) // m * m


@functools.partial(jax.jit, static_argnames=("n_rows", "dim"))
def _gather_rows(table2d, idx_pad, n_rows, dim):
    span = n_rows // _NW
    rem = n_rows - span * _NW
    rem_pad = _ceil_to(rem, 8)
    n_chunks = -(-span // _CH)
    offs = [min(c * _CH, span - _CH) for c in range(n_chunks)]

    mesh = plsc.VectorSubcoreMesh(
        core_axis_name="c", subcore_axis_name="s",
        num_cores=_NC, num_subcores=_NS)

    scratch = [
        pltpu.VMEM((span,), jnp.int32),
        pltpu.VMEM((_CH, dim), table2d.dtype),
        pltpu.VMEM((_CH, dim), table2d.dtype),
        pltpu.SemaphoreType.DMA,
        pltpu.SemaphoreType.DMA,
    ]
    if rem:
        scratch += [
            pltpu.VMEM((rem_pad,), jnp.int32),
            pltpu.VMEM((rem_pad, dim), table2d.dtype),
        ]

    @functools.partial(
        pl.kernel,
        out_type=jax.ShapeDtypeStruct((n_rows, dim), table2d.dtype),
        mesh=mesh,
        scratch_types=scratch,
    )
    def run(table_hbm, idx_hbm, out_hbm, idx_v, rows0, rows1, sem0, sem1,
            *rem_scratch):
        wid = lax.axis_index("s") * _NC + lax.axis_index("c")
        base = wid * span
        rows = (rows0, rows1)
        sems = (sem0, sem1)

        pltpu.sync_copy(idx_hbm.at[pl.ds(base, span)], idx_v)

        def start_gather(c, b):
            pltpu.async_copy(
                table_hbm.at[idx_v.at[pl.ds(offs[c], _CH)]], rows[b], sems[b])

        start_gather(0, 0)
        for i in range(n_chunks):
            b = i & 1
            pltpu.make_async_copy(
                table_hbm.at[idx_v.at[pl.ds(offs[i], _CH)]], rows[b],
                sems[b]).wait()
            if i + 1 < n_chunks:
                start_gather(i + 1, b ^ 1)
            pltpu.sync_copy(rows[b], out_hbm.at[pl.ds(base + offs[i], _CH)])

        if rem:
            idx_t, rows_t = rem_scratch
            tbase = span * _NW

            @pl.when(wid == _NW - 1)
            def _():
                pltpu.sync_copy(idx_hbm.at[pl.ds(tbase, rem_pad)], idx_t)
                pltpu.async_copy(table_hbm.at[idx_t], rows_t, sem0).wait()
                pltpu.sync_copy(rows_t.at[pl.ds(0, rem)],
                                out_hbm.at[pl.ds(tbase, rem)])

    return run(table2d, idx_pad)


def kernel(unique_params, index_map):
    n, in_dim, out_dim = unique_params.shape
    dim = in_dim * out_dim
    b = index_map.size
    table2d = unique_params.reshape(n, dim)
    idx = index_map.reshape(-1).astype(jnp.int32)
    b_pad = _ceil_to(b, 8)
    if b_pad != b:
        idx = jnp.pad(idx, (0, b_pad - b))
    out2d = _gather_rows(table2d, idx, n_rows=b, dim=dim)
    return out2d.reshape(*index_map.shape, in_dim, out_dim)
